# baseline (speedup 1.0000x reference)
import jax
import jax.numpy as jnp
from jax.experimental import pallas as pl
from jax.experimental.pallas import tpu as pltpu

_SUPPORTS_MIN = -300.0
_NUM_SUPPORTS = 601
_EPSILON = 0.001
_ROW_BLOCK = 64


def _two_hot_kernel(x_ref, o_ref):
    x = x_ref[:]
    t = jnp.sign(x) * (jnp.sqrt(jnp.abs(x) + 1.0) - 1.0 + _EPSILON * x)
    lower = jnp.clip(jnp.floor(t - _SUPPORTS_MIN), 0.0,
                     float(_NUM_SUPPORTS - 2)).astype(jnp.int32)
    upper_support = (lower + 1).astype(jnp.float32) + _SUPPORTS_MIN
    p_low = upper_support - t
    p_high = 1.0 - p_low
    rb, k = x.shape
    iota = jax.lax.broadcasted_iota(jnp.int32, (rb, k, _NUM_SUPPORTS), 2)
    lw = lower[:, :, None]
    o_ref[:] = jnp.where(
        iota == lw, p_low[:, :, None],
        jnp.where(iota == lw + 1, p_high[:, :, None], 0.0))


@jax.jit
def kernel(target_value):
    b, k = target_value.shape
    rb = _ROW_BLOCK if b % _ROW_BLOCK == 0 else 1
    return pl.pallas_call(
        _two_hot_kernel,
        grid=(b // rb,),
        in_specs=[pl.BlockSpec((rb, k), lambda i: (i, 0))],
        out_specs=pl.BlockSpec((rb, k, _NUM_SUPPORTS), lambda i: (i, 0, 0)),
        out_shape=jax.ShapeDtypeStruct((b, k, _NUM_SUPPORTS), jnp.float32),
        compiler_params=pltpu.CompilerParams(dimension_semantics=("parallel",)),
    )(target_value)


# speedup vs baseline: 1.0085x; 1.0085x over previous
import jax
import jax.numpy as jnp
from jax.experimental import pallas as pl
from jax.experimental.pallas import tpu as pltpu

_SUPPORTS_MIN = -300.0
_NUM_SUPPORTS = 601
_EPSILON = 0.001
_RB = 32
_SLOTS = 8


def _two_hot_kernel(x_ref, o_hbm, scratch, sems):
    i = pl.program_id(0)
    n = pl.num_programs(0)
    s = jax.lax.rem(i, _SLOTS)
    k = x_ref.shape[1]

    @pl.when(i >= _SLOTS)
    def _wait_prev():
        pltpu.make_async_copy(
            scratch.at[s],
            o_hbm.at[pl.ds(jnp.maximum(i - _SLOTS, 0) * _RB, _RB)],
            sems.at[s]).wait()

    x = x_ref[...]
    t = jnp.sign(x) * (jnp.sqrt(jnp.abs(x) + 1.0) - 1.0 + _EPSILON * x)
    c = jnp.clip(t - _SUPPORTS_MIN, 0.0, float(_NUM_SUPPORTS - 1))
    jf = jax.lax.broadcasted_iota(
        jnp.int32, (_RB, k, _NUM_SUPPORTS), 2).astype(jnp.float32)
    scratch[s] = jnp.maximum(0.0, 1.0 - jnp.abs(jf - c[:, :, None]))

    pltpu.make_async_copy(
        scratch.at[s], o_hbm.at[pl.ds(i * _RB, _RB)], sems.at[s]).start()

    @pl.when(i == n - 1)
    def _drain():
        for j in range(_SLOTS):
            step = n - _SLOTS + j
            pltpu.make_async_copy(
                scratch.at[jax.lax.rem(step, _SLOTS)],
                o_hbm.at[pl.ds(step * _RB, _RB)],
                sems.at[jax.lax.rem(step, _SLOTS)]).wait()


@jax.jit
def kernel(target_value):
    b, k = target_value.shape
    return pl.pallas_call(
        _two_hot_kernel,
        grid=(b // _RB,),
        in_specs=[pl.BlockSpec((_RB, k), lambda i: (i, 0))],
        out_specs=pl.BlockSpec(memory_space=pl.ANY),
        out_shape=jax.ShapeDtypeStruct((b, k, _NUM_SUPPORTS), jnp.float32),
        scratch_shapes=[
            pltpu.VMEM((_SLOTS, _RB, k, _NUM_SUPPORTS), jnp.float32),
            pltpu.SemaphoreType.DMA((_SLOTS,)),
        ],
    )(target_value)


# speedup vs baseline: 1.0172x; 1.0086x over previous
import jax
import jax.numpy as jnp
from jax.experimental import pallas as pl
from jax.experimental.pallas import tpu as pltpu

_SUPPORTS_MIN = -300.0
_NUM_SUPPORTS = 601
_EPSILON = 0.001
_RB = 32
_SLOTS = 8


def _two_hot_kernel(x_ref, o_hbm, scratch, sems):
    i = pl.program_id(0)
    n = pl.num_programs(0)
    s = jax.lax.rem(i, _SLOTS)
    k = x_ref.shape[1]

    @pl.when(i >= _SLOTS)
    def _wait_prev():
        pltpu.make_async_copy(
            scratch.at[s],
            o_hbm.at[pl.ds(jnp.maximum(i - _SLOTS, 0) * _RB, _RB)],
            sems.at[s]).wait()

    x = x_ref[...]
    t = jnp.sign(x) * (jnp.sqrt(jnp.abs(x) + 1.0) - 1.0 + _EPSILON * x)
    c = jnp.clip(t - _SUPPORTS_MIN, 0.0, float(_NUM_SUPPORTS - 1))
    jf = jax.lax.broadcasted_iota(
        jnp.int32, (_RB, k, _NUM_SUPPORTS), 2).astype(jnp.float32)
    scratch[s] = jnp.maximum(0.0, 1.0 - jnp.abs(jf - c[:, :, None]))

    dst = o_hbm.at[pl.ds(i * _RB, _RB)]

    @pl.when(jax.lax.rem(i, 2) == 0)
    def _start_even():
        pltpu.async_copy(scratch.at[s], dst, sems.at[s], priority=0)

    @pl.when(jax.lax.rem(i, 2) == 1)
    def _start_odd():
        pltpu.async_copy(scratch.at[s], dst, sems.at[s], priority=1)

    @pl.when(i == n - 1)
    def _drain():
        for j in range(_SLOTS):
            step = n - _SLOTS + j
            pltpu.make_async_copy(
                scratch.at[jax.lax.rem(step, _SLOTS)],
                o_hbm.at[pl.ds(step * _RB, _RB)],
                sems.at[jax.lax.rem(step, _SLOTS)]).wait()


@jax.jit
def kernel(target_value):
    b, k = target_value.shape
    return pl.pallas_call(
        _two_hot_kernel,
        grid=(b // _RB,),
        in_specs=[pl.BlockSpec((_RB, k), lambda i: (i, 0))],
        out_specs=pl.BlockSpec(memory_space=pl.ANY),
        out_shape=jax.ShapeDtypeStruct((b, k, _NUM_SUPPORTS), jnp.float32),
        scratch_shapes=[
            pltpu.VMEM((_SLOTS, _RB, k, _NUM_SUPPORTS), jnp.float32),
            pltpu.SemaphoreType.DMA((_SLOTS,)),
        ],
    )(target_value)


# speedup vs baseline: 1.0174x; 1.0002x over previous
import jax
import jax.numpy as jnp
from jax.experimental import pallas as pl
from jax.experimental.pallas import tpu as pltpu

_SUPPORTS_MIN = -300.0
_NUM_SUPPORTS = 601
_EPSILON = 0.001
_RB = 32
_SLOTS = 8
_QUEUES = 4
_QR = _RB // _QUEUES


def _two_hot_kernel(x_ref, o_hbm, scratch, sems):
    i = pl.program_id(0)
    n = pl.num_programs(0)
    s = jax.lax.rem(i, _SLOTS)
    k = x_ref.shape[1]

    @pl.when(i >= _SLOTS)
    def _wait_prev():
        prev = jnp.maximum(i - _SLOTS, 0) * _RB
        for q in range(_QUEUES):
            pltpu.make_async_copy(
                scratch.at[s, pl.ds(q * _QR, _QR)],
                o_hbm.at[pl.ds(prev + q * _QR, _QR)],
                sems.at[q, s]).wait()

    x = x_ref[...]
    t = jnp.sign(x) * (jnp.sqrt(jnp.abs(x) + 1.0) - 1.0 + _EPSILON * x)
    c = jnp.clip(t - _SUPPORTS_MIN, 0.0, float(_NUM_SUPPORTS - 1))
    jf = jax.lax.broadcasted_iota(
        jnp.int32, (_RB, k, _NUM_SUPPORTS), 2).astype(jnp.float32)
    scratch[s] = jnp.maximum(0.0, 1.0 - jnp.abs(jf - c[:, :, None]))

    for q in range(_QUEUES):
        pltpu.make_async_copy(
            scratch.at[s, pl.ds(q * _QR, _QR)],
            o_hbm.at[pl.ds(i * _RB + q * _QR, _QR)],
            sems.at[q, s]).start()

    @pl.when(i == n - 1)
    def _drain():
        for j in range(_SLOTS):
            step = n - _SLOTS + j
            for q in range(_QUEUES):
                pltpu.make_async_copy(
                    scratch.at[jax.lax.rem(step, _SLOTS), pl.ds(q * _QR, _QR)],
                    o_hbm.at[pl.ds(step * _RB + q * _QR, _QR)],
                    sems.at[q, jax.lax.rem(step, _SLOTS)]).wait()


@jax.jit
def kernel(target_value):
    b, k = target_value.shape
    return pl.pallas_call(
        _two_hot_kernel,
        grid=(b // _RB,),
        in_specs=[pl.BlockSpec((_RB, k), lambda i: (i, 0))],
        out_specs=pl.BlockSpec(memory_space=pl.ANY),
        out_shape=jax.ShapeDtypeStruct((b, k, _NUM_SUPPORTS), jnp.float32),
        scratch_shapes=[
            pltpu.VMEM((_SLOTS, _RB, k, _NUM_SUPPORTS), jnp.float32),
            pltpu.SemaphoreType.DMA((_QUEUES, _SLOTS)),
        ],
    )(target_value)


# speedup vs baseline: 4.4871x; 4.4104x over previous
import jax
import jax.numpy as jnp
from jax.experimental import pallas as pl

_SUPPORTS_MIN = -300.0
_NUM_SUPPORTS = 601
_EPSILON = 0.001


def _two_hot_kernel(x_ref, o_ref):
    x = x_ref[...]
    t = jnp.sign(x) * (jnp.sqrt(jnp.abs(x) + 1.0) - 1.0 + _EPSILON * x)
    c = jnp.clip(t - _SUPPORTS_MIN, 0.0, float(_NUM_SUPPORTS - 1))
    jf = jax.lax.broadcasted_iota(
        jnp.int32, (_NUM_SUPPORTS, x.shape[1]), 0).astype(jnp.float32)
    o_ref[...] = jnp.maximum(0.0, 1.0 - jnp.abs(jf - c))


@jax.jit
def kernel(target_value):
    b, k = target_value.shape
    xt = target_value.T.reshape(k, 1, b)
    out_p = pl.pallas_call(
        _two_hot_kernel,
        grid=(k,),
        in_specs=[pl.BlockSpec((None, 1, b), lambda i: (i, 0, 0))],
        out_specs=pl.BlockSpec((None, _NUM_SUPPORTS, b), lambda i: (i, 0, 0)),
        out_shape=jax.ShapeDtypeStruct((k, _NUM_SUPPORTS, b), jnp.float32),
    )(xt)
    return out_p.transpose(2, 0, 1)
